# baseline (speedup 1.0000x reference)
import functools

import jax
import jax.numpy as jnp
from jax import lax
from jax.experimental import pallas as pl
from jax.experimental.pallas import tpu as pltpu
from jax.experimental.pallas import tpu_sc as plsc

NN = 10000
EE = 320000
GG = 128
NC = 2
NS = 16
NWK = NC * NS
NPT = 320
NP = NWK * NPT
EP = 327680
WG = EP // NWK
CG = 256
SCHUNK = 3200
STG = SCHUNK + 32

@functools.lru_cache(maxsize=None)
def _mesh():
    return plsc.VectorSubcoreMesh(core_axis_name="c", subcore_axis_name="s")


def _wid():
    return lax.axis_index("c") * NS + lax.axis_index("s")


def _sc_count_body(dst_hbm, counts_hbm, dbuf, cscr):
    wid = _wid()
    lo = wid * NPT
    hi = lo + NPT

    cscr[...] = jnp.zeros((16,), jnp.int32)
    lov = jnp.broadcast_to(lo, (16,))
    nptv = jnp.full((16,), NPT, jnp.uint32)
    onev = jnp.ones((16,), jnp.int32)
    zerov = jnp.zeros((16,), jnp.int32)

    def chunk(k, _):
        pltpu.sync_copy(dst_hbm.at[pl.ds(k * SCHUNK, SCHUNK)], dbuf)

        def inner(j, _2):
            d = dbuf[pl.ds(j * 16, 16)]
            m = plsc.bitcast(d - lov, jnp.uint32) < nptv
            cscr[...] = cscr[...] + jnp.where(m, onev, zerov)
            return 0

        lax.fori_loop(0, SCHUNK // 16, inner, 0)
        return 0

    lax.fori_loop(0, EE // SCHUNK, chunk, 0)
    pltpu.sync_copy(cscr, counts_hbm.at[wid])


@functools.lru_cache(maxsize=None)
def _sc_count():
    return pl.kernel(
        _sc_count_body, mesh=_mesh(),
        out_type=jax.ShapeDtypeStruct((NWK, 16), jnp.int32),
        scratch_types=[pltpu.VMEM((SCHUNK,), jnp.int32),
                       pltpu.VMEM((16,), jnp.int32)])


def _prefix16(x):
    lanev = lax.broadcasted_iota(jnp.int32, (16,), 0)
    zerov = jnp.zeros((16,), jnp.int32)
    dn = lax.GatherDimensionNumbers(offset_dims=(), collapsed_slice_dims=(0,),
                                    start_index_map=(0,))
    for sh in (1, 2, 4, 8):
        idxs = jnp.maximum(lanev - sh, zerov)
        shifted = lax.gather(x, idxs[:, None], dn, (1,),
                             mode=lax.GatherScatterMode.PROMISE_IN_BOUNDS)
        x = x + jnp.where(lanev >= jnp.full((16,), sh, jnp.int32), shifted,
                          zerov)
    return x


def _sc_bucket_body(dst_hbm, src_hbm, offs_hbm, dstp_hbm, srcp_hbm,
                    dbuf, sbuf, vidx, vvd, vvs, orow, sem):
    wid = _wid()
    lo = wid * NPT
    pltpu.sync_copy(offs_hbm.at[wid], orow)
    base = orow[...][0]
    lov = jnp.broadcast_to(lo, (16,))
    nptv = jnp.full((16,), NPT, jnp.uint32)
    onev = jnp.ones((16,), jnp.int32)
    zerov = jnp.zeros((16,), jnp.int32)
    negv = jnp.full((16,), -1, jnp.int32)
    lanev = lax.broadcasted_iota(jnp.int32, (16,), 0)
    trashv = jnp.full((16,), EP - 16, jnp.int32) + lanev

    def chunk(k, pos0):
        pltpu.sync_copy(dst_hbm.at[pl.ds(k * SCHUNK, SCHUNK)], dbuf)
        pltpu.sync_copy(src_hbm.at[pl.ds(k * SCHUNK, SCHUNK)], sbuf)

        def inner(j, pos):
            jo = pl.multiple_of(j * 16, 16)
            d = dbuf[pl.ds(jo, 16)]
            s = sbuf[pl.ds(jo, 16)]
            m = plsc.bitcast(d - lov, jnp.uint32) < nptv
            m_i32 = jnp.where(m, onev, zerov)
            incl = _prefix16(m_i32)
            excl = incl - m_i32
            tgt = jnp.broadcast_to(pos, (16,)) + excl
            idx = m_i32 * tgt + (onev - m_i32) * trashv
            row = j // 8
            cb = pl.multiple_of((j % 8) * 16, 16)
            vidx[row, pl.ds(cb, 16)] = idx
            vvd[row, pl.ds(cb, 16)] = d
            vvs[row, pl.ds(cb, 16)] = s
            return pos + incl[15]

        pos = lax.fori_loop(0, SCHUNK // 16, inner, pos0)

        def scat(r, _2):
            pltpu.async_copy(vvd.at[r], dstp_hbm.at[vidx.at[r]], sem).wait()
            pltpu.async_copy(vvs.at[r], srcp_hbm.at[vidx.at[r]], sem).wait()
            return 0

        lax.fori_loop(0, SCHUNK // 128, scat, 0)
        return pos

    pos = lax.fori_loop(0, EE // SCHUNK, chunk, base)
    vidx[0, pl.ds(0, 16)] = jnp.broadcast_to(pos, (16,)) + lanev
    for c in range(1, 8):
        vidx[0, pl.ds(c * 16, 16)] = trashv
    for c in range(8):
        vvd[0, pl.ds(c * 16, 16)] = negv
        vvs[0, pl.ds(c * 16, 16)] = negv
    pltpu.async_copy(vvd.at[0], dstp_hbm.at[vidx.at[0]], sem).wait()
    pltpu.async_copy(vvs.at[0], srcp_hbm.at[vidx.at[0]], sem).wait()


@functools.lru_cache(maxsize=None)
def _sc_bucket():
    return pl.kernel(
        _sc_bucket_body, mesh=_mesh(),
        out_type=(jax.ShapeDtypeStruct((EP,), jnp.int32),
                  jax.ShapeDtypeStruct((EP,), jnp.int32)),
        scratch_types=[pltpu.VMEM((SCHUNK,), jnp.int32),
                       pltpu.VMEM((SCHUNK,), jnp.int32),
                       pltpu.VMEM((SCHUNK // 128, 128), jnp.int32),
                       pltpu.VMEM((SCHUNK // 128, 128), jnp.int32),
                       pltpu.VMEM((SCHUNK // 128, 128), jnp.int32),
                       pltpu.VMEM((16,), jnp.int32),
                       pltpu.SemaphoreType.DMA])


@functools.lru_cache(maxsize=None)
def _make_sc_gather(nrows):
    def gather(pq_hbm, dstp2, srcp2, v_hbm, di, si, pb, qb, vb, sp, sq):
        wid = _wid()
        ziv = jnp.zeros((16,), jnp.int32)
        mxv = jnp.full((16,), nrows - 1, jnp.int32)
        zfv = jnp.zeros((16,), jnp.float32)

        def chunk(k, _):
            r0 = wid * WG + k * CG
            pltpu.sync_copy(dstp2.at[pl.ds(wid * (WG // 128) + k * 2, 2)], di)
            pltpu.sync_copy(srcp2.at[pl.ds(wid * (WG // 128) + k * 2, 2)], si)
            for j in range(2):
                for c in range(8):
                    dvv = di[j, pl.ds(c * 16, 16)]
                    di[j, pl.ds(c * 16, 16)] = jnp.minimum(
                        jnp.maximum(dvv, ziv), mxv)
                    svv = si[j, pl.ds(c * 16, 16)]
                    si[j, pl.ds(c * 16, 16)] = jnp.minimum(
                        jnp.maximum(svv, ziv), mxv)
            cps = [pltpu.async_copy(pq_hbm.at[di.at[j]],
                                    pb.at[pl.ds(j * 128, 128)], sp)
                   for j in range(2)]
            cqs = [pltpu.async_copy(pq_hbm.at[si.at[j]],
                                    qb.at[pl.ds(j * 128, 128)], sq)
                   for j in range(2)]
            for cp in cps:
                cp.wait()
            for cq in cqs:
                cq.wait()

            def row(r, _2):
                for c in range(4):
                    vb[r, pl.ds(c * 16, 16)] = jnp.maximum(
                        pb[r, pl.ds(c * 16, 16)]
                        + qb[r, pl.ds(64 + c * 16, 16)], zfv)
                return 0

            lax.fori_loop(0, CG, row, 0)
            pltpu.sync_copy(vb, v_hbm.at[pl.ds(r0, CG)])
            return 0

        lax.fori_loop(0, WG // CG, chunk, 0)

    return pl.kernel(
        gather, mesh=_mesh(),
        out_type=jax.ShapeDtypeStruct((EP, 64), jnp.float32),
        scratch_types=[pltpu.VMEM((2, 128), jnp.int32),
                       pltpu.VMEM((2, 128), jnp.int32),
                       pltpu.VMEM((CG, 128), jnp.float32),
                       pltpu.VMEM((CG, 128), jnp.float32),
                       pltpu.VMEM((CG, 64), jnp.float32),
                       pltpu.SemaphoreType.DMA,
                       pltpu.SemaphoreType.DMA])


def _sc_max_body(m_hbm, dstp_hbm, offs_hbm, cnts_hbm, h_hbm,
                 acc, mb, dv, offv, cntv):
    wid = _wid()
    lo = wid * NPT
    pltpu.sync_copy(offs_hbm.at[wid], offv)
    pltpu.sync_copy(cnts_hbm.at[wid], cntv)
    off = pl.multiple_of(offv[...][0], 16)
    cnt = cntv[...][0]

    def zr(r, _):
        for c in range(4):
            acc[r, pl.ds(c * 16, 16)] = jnp.zeros((16,), jnp.float32)
        return 0

    lax.fori_loop(0, NPT + 8, zr, 0)
    nch = (cnt + CG - 1) // CG

    def chunk(k, _):
        pltpu.sync_copy(dstp_hbm.at[pl.ds(off + k * CG, CG)], dv)
        pltpu.sync_copy(m_hbm.at[pl.ds(off + k * CG, CG)], mb)
        ne = jnp.minimum(CG, cnt - k * CG)
        nb16 = (ne + 15) // 16

        lov = jnp.broadcast_to(lo, (16,))

        def blk(b, _2):
            dvec = dv[pl.ds(b * 16, 16)] - lov
            for lane in range(16):
                dl = dvec[lane]
                dls = jnp.where(dl >= 0, dl, NPT)
                e = b * 16 + lane
                for c in range(4):
                    acc[dls, pl.ds(c * 16, 16)] = jnp.maximum(
                        acc[dls, pl.ds(c * 16, 16)], mb[e, pl.ds(c * 16, 16)])
            return 0

        lax.fori_loop(0, nb16, blk, 0)
        return 0

    lax.fori_loop(0, nch, chunk, 0)
    pltpu.sync_copy(acc.at[pl.ds(0, NPT)], h_hbm.at[pl.ds(lo, NPT)])


@functools.lru_cache(maxsize=None)
def _sc_max():
    return pl.kernel(
        _sc_max_body, mesh=_mesh(),
        out_type=jax.ShapeDtypeStruct((NP, 64), jnp.float32),
        scratch_types=[pltpu.VMEM((NPT + 8, 64), jnp.float32),
                       pltpu.VMEM((CG, 64), jnp.float32),
                       pltpu.VMEM((CG,), jnp.int32),
                       pltpu.VMEM((16,), jnp.int32),
                       pltpu.VMEM((16,), jnp.int32)])


def _proj_body(x_ref, wm_ref, wb_ref, b_ref, pq_ref):
    xb = x_ref[...]
    pq_ref[:, 0:64] = jnp.dot(xb, wm_ref[...],
                              preferred_element_type=jnp.float32) + b_ref[...]
    pq_ref[:, 64:128] = jnp.dot(xb, wb_ref[...],
                                preferred_element_type=jnp.float32)


def _tc_proj(xin, wm, wb, b1):
    rows, fdim = xin.shape
    br = 1000 if rows == NN else 1024
    return pl.pallas_call(
        _proj_body,
        grid=(rows // br,),
        in_specs=[pl.BlockSpec((br, fdim), lambda i: (i, 0)),
                  pl.BlockSpec((fdim, 64), lambda i: (0, 0)),
                  pl.BlockSpec((fdim, 64), lambda i: (0, 0)),
                  pl.BlockSpec((1, 64), lambda i: (0, 0))],
        out_specs=pl.BlockSpec((br, 128), lambda i: (i, 0)),
        out_shape=jax.ShapeDtypeStruct((rows, 128), jnp.float32),
    )(xin, wm, wb, b1.reshape(1, 64))


def _mm_body(v_ref, w_ref, b_ref, m_ref):
    m_ref[...] = jnp.dot(v_ref[...], w_ref[...],
                         preferred_element_type=jnp.float32) + b_ref[...]


def _tc_mm(v, w2, b2):
    br = 1024
    return pl.pallas_call(
        _mm_body,
        grid=(EP // br,),
        in_specs=[pl.BlockSpec((br, 64), lambda i: (i, 0)),
                  pl.BlockSpec((64, 64), lambda i: (0, 0)),
                  pl.BlockSpec((1, 64), lambda i: (0, 0))],
        out_specs=pl.BlockSpec((br, 64), lambda i: (i, 0)),
        out_shape=jax.ShapeDtypeStruct((EP, 64), jnp.float32),
    )(v, w2, b2.reshape(1, 64))


def _pool_body(h_ref, b_ref, wf1_ref, bf1_ref, wf2_ref, bf2_ref, o_ref,
               acc, cacc):
    i = pl.program_id(0)

    @pl.when(i == 0)
    def _():
        acc[...] = jnp.zeros_like(acc)
        cacc[...] = jnp.zeros_like(cacc)

    bvec = b_ref[0, 0, :].reshape(1, 1024)
    gids = lax.broadcasted_iota(jnp.int32, (GG, 1024), 0)
    mask = (gids == jnp.broadcast_to(bvec, (GG, 1024))).astype(jnp.float32)
    acc[...] += jnp.dot(mask, h_ref[...], preferred_element_type=jnp.float32)
    cacc[...] += jnp.sum(mask, axis=1, keepdims=True)

    @pl.when(i == (NP // 1024) - 1)
    def _():
        pooled = acc[...] / jnp.maximum(cacc[...], 1.0)
        o1 = jnp.maximum(
            jnp.dot(pooled, wf1_ref[...],
                    preferred_element_type=jnp.float32) + bf1_ref[...], 0.0)
        o_ref[...] = jnp.dot(o1, wf2_ref[...],
                             preferred_element_type=jnp.float32) + bf2_ref[...]


def _tc_pool(h3, batch3, wf1, bf1, wf2p, bf2p):
    return pl.pallas_call(
        _pool_body,
        grid=(NP // 1024,),
        in_specs=[pl.BlockSpec((1024, 64), lambda i: (i, 0)),
                  pl.BlockSpec((1, 1, 1024), lambda i: (i, 0, 0)),
                  pl.BlockSpec((64, 64), lambda i: (0, 0)),
                  pl.BlockSpec((1, 64), lambda i: (0, 0)),
                  pl.BlockSpec((64, 128), lambda i: (0, 0)),
                  pl.BlockSpec((1, 128), lambda i: (0, 0))],
        out_specs=pl.BlockSpec((GG, 128), lambda i: (0, 0)),
        out_shape=jax.ShapeDtypeStruct((GG, 128), jnp.float32),
        scratch_shapes=[pltpu.VMEM((GG, 64), jnp.float32),
                        pltpu.VMEM((GG, 1), jnp.float32)],
    )(h3, batch3, wf1, bf1.reshape(1, 64), wf2p, bf2p.reshape(1, 128))


def kernel(x, edge_index, batch, W1a, b1a, W2a, b2a, W1b, b1b, W2b, b2b,
           W1c, b1c, W2c, b2c, Wf1, bf1, Wf2, bf2):
    src = edge_index[0]
    dst = edge_index[1]

    counts2d = _sc_count()(dst)
    cnt32 = jnp.sum(counts2d, axis=1)
    pc = ((cnt32 + 15) // 16 + 1) * 16
    offs = jnp.concatenate(
        [jnp.zeros((1,), jnp.int32),
         jnp.cumsum(pc)[:-1].astype(jnp.int32)])
    offs_b = jnp.broadcast_to(offs[:, None], (NWK, 16))
    cnts_b = jnp.broadcast_to(cnt32[:, None], (NWK, 16)).astype(jnp.int32)
    dst_p, src_p = _sc_bucket()(dst, src, offs_b)
    dst_p2 = dst_p.reshape(EP // 128, 128)
    src_p2 = src_p.reshape(EP // 128, 128)

    def layer(xin, w1, b1, w2, b2, gather_fn):
        fdim = xin.shape[1]
        wm = w1[:fdim] - w1[fdim:]
        wb = w1[fdim:]
        pq = _tc_proj(xin, wm, wb, b1)
        v = gather_fn(pq, dst_p2, src_p2)
        m = _tc_mm(v, w2, b2)
        return _sc_max()(m, dst_p, offs_b, cnts_b)

    h1 = layer(x, W1a, b1a, W2a, b2a, _make_sc_gather(NN))
    h2 = layer(h1, W1b, b1b, W2b, b2b, _make_sc_gather(NP))
    h3 = layer(h2, W1c, b1c, W2c, b2c, _make_sc_gather(NP))

    batch_p = jnp.concatenate(
        [batch, jnp.full((NP - NN,), GG, jnp.int32)]).reshape(NP // 1024, 1, 1024)
    wf2p = jnp.zeros((64, 128), jnp.float32).at[:, :2].set(Wf2)
    bf2p = jnp.zeros((128,), jnp.float32).at[:2].set(bf2)
    out = _tc_pool(h3, batch_p, Wf1, bf1, wf2p, bf2p)
    return out[:, :2]


# speedup vs baseline: 257.0295x; 257.0295x over previous
import functools

import jax
import jax.numpy as jnp
from jax import lax
from jax.experimental import pallas as pl
from jax.experimental.pallas import tpu as pltpu
from jax.experimental.pallas import tpu_sc as plsc

NN = 10000
EE = 320000
GG = 128
NC = 2
NS = 16
NWK = NC * NS
NPT = 320
NP = NWK * NPT
EP = 327680
WG = EP // NWK
CG = 256
SCHUNK = 3200
STG = SCHUNK + 32

@functools.lru_cache(maxsize=None)
def _mesh():
    return plsc.VectorSubcoreMesh(core_axis_name="c", subcore_axis_name="s")


def _wid():
    return lax.axis_index("c") * NS + lax.axis_index("s")


def _sc_count_body(dst_hbm, counts_hbm, dbuf, cscr):
    wid = _wid()
    lo = wid * NPT
    hi = lo + NPT

    cscr[...] = jnp.zeros((16,), jnp.int32)
    lov = jnp.broadcast_to(lo, (16,))
    nptv = jnp.full((16,), NPT, jnp.uint32)
    onev = jnp.ones((16,), jnp.int32)
    zerov = jnp.zeros((16,), jnp.int32)

    def chunk(k, _):
        pltpu.sync_copy(dst_hbm.at[pl.ds(k * SCHUNK, SCHUNK)], dbuf)

        def inner(j, _2):
            d = dbuf[pl.ds(j * 16, 16)]
            m = plsc.bitcast(d - lov, jnp.uint32) < nptv
            cscr[...] = cscr[...] + jnp.where(m, onev, zerov)
            return 0

        lax.fori_loop(0, SCHUNK // 16, inner, 0)
        return 0

    lax.fori_loop(0, EE // SCHUNK, chunk, 0)
    pltpu.sync_copy(cscr, counts_hbm.at[wid])


@functools.lru_cache(maxsize=None)
def _sc_count():
    return pl.kernel(
        _sc_count_body, mesh=_mesh(),
        out_type=jax.ShapeDtypeStruct((NWK, 16), jnp.int32),
        scratch_types=[pltpu.VMEM((SCHUNK,), jnp.int32),
                       pltpu.VMEM((16,), jnp.int32)])


def _prefix16(x):
    lanev = lax.broadcasted_iota(jnp.int32, (16,), 0)
    zerov = jnp.zeros((16,), jnp.int32)
    dn = lax.GatherDimensionNumbers(offset_dims=(), collapsed_slice_dims=(0,),
                                    start_index_map=(0,))
    for sh in (1, 2, 4, 8):
        idxs = jnp.maximum(lanev - sh, zerov)
        shifted = lax.gather(x, idxs[:, None], dn, (1,),
                             mode=lax.GatherScatterMode.PROMISE_IN_BOUNDS)
        x = x + jnp.where(lanev >= jnp.full((16,), sh, jnp.int32), shifted,
                          zerov)
    return x


SPER = 4096


def _sc_bucket_body(dst_hbm, src_hbm, offs_hbm, dstp_hbm, srcp_hbm,
                    dbuf, sbuf, vidx, vvd, vvs, tmp, orow, shd, shs, sem):
    wid = _wid()
    sid = lax.axis_index("s")
    sidbase = sid * SPER
    sbm = pl.multiple_of(sidbase, 16)
    lo = wid * NPT
    pltpu.sync_copy(offs_hbm.at[wid], orow)
    base = pl.multiple_of(orow[...][0], 16)
    lov = jnp.broadcast_to(lo, (16,))
    nptv = jnp.full((16,), NPT, jnp.uint32)
    onev = jnp.ones((16,), jnp.int32)
    zerov = jnp.zeros((16,), jnp.int32)
    negv = jnp.full((16,), -1, jnp.int32)
    lanev = lax.broadcasted_iota(jnp.int32, (16,), 0)
    sb16 = jnp.broadcast_to(sidbase, (16,))
    trashv = sb16 + jnp.full((16,), SPER - 16, jnp.int32) + lanev

    def chunk(k, carry):
        rel0, wp0 = carry
        pltpu.sync_copy(dst_hbm.at[pl.ds(k * SCHUNK, SCHUNK)], dbuf)
        pltpu.sync_copy(src_hbm.at[pl.ds(k * SCHUNK, SCHUNK)], sbuf)

        def inner(j, rel):
            jo = pl.multiple_of(j * 16, 16)
            d = dbuf[pl.ds(jo, 16)]
            s = sbuf[pl.ds(jo, 16)]
            m = plsc.bitcast(d - lov, jnp.uint32) < nptv
            m_i32 = jnp.where(m, onev, zerov)
            incl = _prefix16(m_i32)
            excl = incl - m_i32
            tgt = sb16 + jnp.broadcast_to(rel, (16,)) + excl
            idx = m_i32 * tgt + (onev - m_i32) * trashv
            row = j // 8
            cb = pl.multiple_of((j % 8) * 16, 16)
            vidx[row, pl.ds(cb, 16)] = idx
            vvd[row, pl.ds(cb, 16)] = d
            vvs[row, pl.ds(cb, 16)] = s
            return rel + incl[15]

        rel = lax.fori_loop(0, SCHUNK // 16, inner, rel0)

        def scat(r, _2):
            pltpu.async_copy(vvd.at[r], shd.at[vidx.at[r]], sem).wait()
            pltpu.async_copy(vvs.at[r], shs.at[vidx.at[r]], sem).wait()
            return 0

        lax.fori_loop(0, SCHUNK // 128, scat, 0)

        nb = rel // 16
        wp = pl.multiple_of(wp0, 16)

        def fl(b, _2):
            pltpu.sync_copy(shd.at[pl.ds(sbm + b * 16, 16)], tmp)
            pltpu.sync_copy(tmp, dstp_hbm.at[pl.ds(wp + b * 16, 16)])
            pltpu.sync_copy(shs.at[pl.ds(sbm + b * 16, 16)], tmp)
            pltpu.sync_copy(tmp, srcp_hbm.at[pl.ds(wp + b * 16, 16)])
            return 0

        lax.fori_loop(0, nb, fl, 0)
        mo = pl.multiple_of(nb * 16, 16)
        pltpu.sync_copy(shd.at[pl.ds(sbm + mo, 16)], tmp)
        pltpu.sync_copy(tmp, shd.at[pl.ds(sbm, 16)])
        pltpu.sync_copy(shs.at[pl.ds(sbm + mo, 16)], tmp)
        pltpu.sync_copy(tmp, shs.at[pl.ds(sbm, 16)])
        return (rel - nb * 16, wp + nb * 16)

    rel, wp0 = lax.fori_loop(0, EE // SCHUNK, chunk, (jnp.int32(0), base))
    wp = pl.multiple_of(wp0, 16)
    relv = jnp.broadcast_to(rel, (16,))
    pltpu.sync_copy(shd.at[pl.ds(sbm, 16)], tmp)
    tmp[...] = jnp.where(lanev < relv, tmp[...], negv)
    pltpu.sync_copy(tmp, dstp_hbm.at[pl.ds(wp, 16)])
    pltpu.sync_copy(shs.at[pl.ds(sbm, 16)], tmp)
    relv2 = jnp.broadcast_to(rel, (16,))
    tmp[...] = jnp.where(lanev < relv2, tmp[...], negv)
    pltpu.sync_copy(tmp, srcp_hbm.at[pl.ds(wp, 16)])


@functools.lru_cache(maxsize=None)
def _sc_bucket():
    return pl.kernel(
        _sc_bucket_body, mesh=_mesh(),
        out_type=(jax.ShapeDtypeStruct((EP,), jnp.int32),
                  jax.ShapeDtypeStruct((EP,), jnp.int32)),
        scratch_types=[pltpu.VMEM((SCHUNK,), jnp.int32),
                       pltpu.VMEM((SCHUNK,), jnp.int32),
                       pltpu.VMEM((SCHUNK // 128, 128), jnp.int32),
                       pltpu.VMEM((SCHUNK // 128, 128), jnp.int32),
                       pltpu.VMEM((SCHUNK // 128, 128), jnp.int32),
                       pltpu.VMEM((16,), jnp.int32),
                       pltpu.VMEM((16,), jnp.int32),
                       pltpu.VMEM_SHARED((NS * SPER,), jnp.int32),
                       pltpu.VMEM_SHARED((NS * SPER,), jnp.int32),
                       pltpu.SemaphoreType.DMA])


@functools.lru_cache(maxsize=None)
def _make_sc_gather(nrows):
    def gather(pq_hbm, dstp2, srcp2, v_hbm, di, si, pb, qb, vb, sp, sq):
        wid = _wid()
        ziv = jnp.zeros((16,), jnp.int32)
        mxv = jnp.full((16,), nrows - 1, jnp.int32)
        zfv = jnp.zeros((16,), jnp.float32)

        def chunk(k, _):
            r0 = wid * WG + k * CG
            pltpu.sync_copy(dstp2.at[pl.ds(wid * (WG // 128) + k * 2, 2)], di)
            pltpu.sync_copy(srcp2.at[pl.ds(wid * (WG // 128) + k * 2, 2)], si)
            for j in range(2):
                for c in range(8):
                    dvv = di[j, pl.ds(c * 16, 16)]
                    di[j, pl.ds(c * 16, 16)] = jnp.minimum(
                        jnp.maximum(dvv, ziv), mxv)
                    svv = si[j, pl.ds(c * 16, 16)]
                    si[j, pl.ds(c * 16, 16)] = jnp.minimum(
                        jnp.maximum(svv, ziv), mxv)
            cps = [pltpu.async_copy(pq_hbm.at[di.at[j]],
                                    pb.at[pl.ds(j * 128, 128)], sp)
                   for j in range(2)]
            cqs = [pltpu.async_copy(pq_hbm.at[si.at[j]],
                                    qb.at[pl.ds(j * 128, 128)], sq)
                   for j in range(2)]
            for cp in cps:
                cp.wait()
            for cq in cqs:
                cq.wait()

            def row(r, _2):
                for c in range(4):
                    vb[r, pl.ds(c * 16, 16)] = jnp.maximum(
                        pb[r, pl.ds(c * 16, 16)]
                        + qb[r, pl.ds(64 + c * 16, 16)], zfv)
                return 0

            lax.fori_loop(0, CG, row, 0)
            pltpu.sync_copy(vb, v_hbm.at[pl.ds(r0, CG)])
            return 0

        lax.fori_loop(0, WG // CG, chunk, 0)

    return pl.kernel(
        gather, mesh=_mesh(),
        out_type=jax.ShapeDtypeStruct((EP, 64), jnp.float32),
        scratch_types=[pltpu.VMEM((2, 128), jnp.int32),
                       pltpu.VMEM((2, 128), jnp.int32),
                       pltpu.VMEM((CG, 128), jnp.float32),
                       pltpu.VMEM((CG, 128), jnp.float32),
                       pltpu.VMEM((CG, 64), jnp.float32),
                       pltpu.SemaphoreType.DMA,
                       pltpu.SemaphoreType.DMA])


def _sc_max_body(m_hbm, dstp_hbm, offs_hbm, cnts_hbm, h_hbm,
                 acc, mb, dv, offv, cntv):
    wid = _wid()
    lo = wid * NPT
    pltpu.sync_copy(offs_hbm.at[wid], offv)
    pltpu.sync_copy(cnts_hbm.at[wid], cntv)
    off = pl.multiple_of(offv[...][0], 16)
    cnt = cntv[...][0]

    def zr(r, _):
        for c in range(4):
            acc[r, pl.ds(c * 16, 16)] = jnp.zeros((16,), jnp.float32)
        return 0

    lax.fori_loop(0, NPT + 8, zr, 0)
    nch = (cnt + CG - 1) // CG

    def chunk(k, _):
        pltpu.sync_copy(dstp_hbm.at[pl.ds(off + k * CG, CG)], dv)
        pltpu.sync_copy(m_hbm.at[pl.ds(off + k * CG, CG)], mb)
        ne = jnp.minimum(CG, cnt - k * CG)
        nb16 = (ne + 15) // 16

        lov = jnp.broadcast_to(lo, (16,))

        def blk(b, _2):
            dvec = dv[pl.ds(b * 16, 16)] - lov
            for lane in range(16):
                dl = dvec[lane]
                dls = jnp.where(dl >= 0, dl, NPT)
                e = b * 16 + lane
                for c in range(4):
                    acc[dls, pl.ds(c * 16, 16)] = jnp.maximum(
                        acc[dls, pl.ds(c * 16, 16)], mb[e, pl.ds(c * 16, 16)])
            return 0

        lax.fori_loop(0, nb16, blk, 0)
        return 0

    lax.fori_loop(0, nch, chunk, 0)
    pltpu.sync_copy(acc.at[pl.ds(0, NPT)], h_hbm.at[pl.ds(lo, NPT)])


@functools.lru_cache(maxsize=None)
def _sc_max():
    return pl.kernel(
        _sc_max_body, mesh=_mesh(),
        out_type=jax.ShapeDtypeStruct((NP, 64), jnp.float32),
        scratch_types=[pltpu.VMEM((NPT + 8, 64), jnp.float32),
                       pltpu.VMEM((CG, 64), jnp.float32),
                       pltpu.VMEM((CG,), jnp.int32),
                       pltpu.VMEM((16,), jnp.int32),
                       pltpu.VMEM((16,), jnp.int32)])


def _proj_body(x_ref, wm_ref, wb_ref, b_ref, pq_ref):
    xb = x_ref[...]
    pq_ref[:, 0:64] = jnp.dot(xb, wm_ref[...],
                              preferred_element_type=jnp.float32) + b_ref[...]
    pq_ref[:, 64:128] = jnp.dot(xb, wb_ref[...],
                                preferred_element_type=jnp.float32)


def _tc_proj(xin, wm, wb, b1):
    rows, fdim = xin.shape
    br = 1000 if rows == NN else 1024
    return pl.pallas_call(
        _proj_body,
        grid=(rows // br,),
        in_specs=[pl.BlockSpec((br, fdim), lambda i: (i, 0)),
                  pl.BlockSpec((fdim, 64), lambda i: (0, 0)),
                  pl.BlockSpec((fdim, 64), lambda i: (0, 0)),
                  pl.BlockSpec((1, 64), lambda i: (0, 0))],
        out_specs=pl.BlockSpec((br, 128), lambda i: (i, 0)),
        out_shape=jax.ShapeDtypeStruct((rows, 128), jnp.float32),
    )(xin, wm, wb, b1.reshape(1, 64))


def _mm_body(v_ref, w_ref, b_ref, m_ref):
    m_ref[...] = jnp.dot(v_ref[...], w_ref[...],
                         preferred_element_type=jnp.float32) + b_ref[...]


def _tc_mm(v, w2, b2):
    br = 1024
    return pl.pallas_call(
        _mm_body,
        grid=(EP // br,),
        in_specs=[pl.BlockSpec((br, 64), lambda i: (i, 0)),
                  pl.BlockSpec((64, 64), lambda i: (0, 0)),
                  pl.BlockSpec((1, 64), lambda i: (0, 0))],
        out_specs=pl.BlockSpec((br, 64), lambda i: (i, 0)),
        out_shape=jax.ShapeDtypeStruct((EP, 64), jnp.float32),
    )(v, w2, b2.reshape(1, 64))


def _pool_body(h_ref, b_ref, wf1_ref, bf1_ref, wf2_ref, bf2_ref, o_ref,
               acc, cacc):
    i = pl.program_id(0)

    @pl.when(i == 0)
    def _():
        acc[...] = jnp.zeros_like(acc)
        cacc[...] = jnp.zeros_like(cacc)

    bvec = b_ref[0, 0, :].reshape(1, 1024)
    gids = lax.broadcasted_iota(jnp.int32, (GG, 1024), 0)
    mask = (gids == jnp.broadcast_to(bvec, (GG, 1024))).astype(jnp.float32)
    acc[...] += jnp.dot(mask, h_ref[...], preferred_element_type=jnp.float32)
    cacc[...] += jnp.sum(mask, axis=1, keepdims=True)

    @pl.when(i == (NP // 1024) - 1)
    def _():
        pooled = acc[...] / jnp.maximum(cacc[...], 1.0)
        o1 = jnp.maximum(
            jnp.dot(pooled, wf1_ref[...],
                    preferred_element_type=jnp.float32) + bf1_ref[...], 0.0)
        o_ref[...] = jnp.dot(o1, wf2_ref[...],
                             preferred_element_type=jnp.float32) + bf2_ref[...]


def _tc_pool(h3, batch3, wf1, bf1, wf2p, bf2p):
    return pl.pallas_call(
        _pool_body,
        grid=(NP // 1024,),
        in_specs=[pl.BlockSpec((1024, 64), lambda i: (i, 0)),
                  pl.BlockSpec((1, 1, 1024), lambda i: (i, 0, 0)),
                  pl.BlockSpec((64, 64), lambda i: (0, 0)),
                  pl.BlockSpec((1, 64), lambda i: (0, 0)),
                  pl.BlockSpec((64, 128), lambda i: (0, 0)),
                  pl.BlockSpec((1, 128), lambda i: (0, 0))],
        out_specs=pl.BlockSpec((GG, 128), lambda i: (0, 0)),
        out_shape=jax.ShapeDtypeStruct((GG, 128), jnp.float32),
        scratch_shapes=[pltpu.VMEM((GG, 64), jnp.float32),
                        pltpu.VMEM((GG, 1), jnp.float32)],
    )(h3, batch3, wf1, bf1.reshape(1, 64), wf2p, bf2p.reshape(1, 128))


def kernel(x, edge_index, batch, W1a, b1a, W2a, b2a, W1b, b1b, W2b, b2b,
           W1c, b1c, W2c, b2c, Wf1, bf1, Wf2, bf2):
    src = edge_index[0]
    dst = edge_index[1]

    counts2d = _sc_count()(dst)
    cnt32 = jnp.sum(counts2d, axis=1)
    pc = ((cnt32 + 15) // 16 + 1) * 16
    offs = jnp.concatenate(
        [jnp.zeros((1,), jnp.int32),
         jnp.cumsum(pc)[:-1].astype(jnp.int32)])
    offs_b = jnp.broadcast_to(offs[:, None], (NWK, 16))
    cnts_b = jnp.broadcast_to(cnt32[:, None], (NWK, 16)).astype(jnp.int32)
    dst_p, src_p = _sc_bucket()(dst, src, offs_b)
    dst_p2 = dst_p.reshape(EP // 128, 128)
    src_p2 = src_p.reshape(EP // 128, 128)

    def layer(xin, w1, b1, w2, b2, gather_fn):
        fdim = xin.shape[1]
        wm = w1[:fdim] - w1[fdim:]
        wb = w1[fdim:]
        pq = _tc_proj(xin, wm, wb, b1)
        v = gather_fn(pq, dst_p2, src_p2)
        m = _tc_mm(v, w2, b2)
        return _sc_max()(m, dst_p, offs_b, cnts_b)

    h1 = layer(x, W1a, b1a, W2a, b2a, _make_sc_gather(NN))
    h2 = layer(h1, W1b, b1b, W2b, b2b, _make_sc_gather(NP))
    h3 = layer(h2, W1c, b1c, W2c, b2c, _make_sc_gather(NP))

    batch_p = jnp.concatenate(
        [batch, jnp.full((NP - NN,), GG, jnp.int32)]).reshape(NP // 1024, 1, 1024)
    wf2p = jnp.zeros((64, 128), jnp.float32).at[:, :2].set(Wf2)
    bf2p = jnp.zeros((128,), jnp.float32).at[:2].set(bf2)
    out = _tc_pool(h3, batch_p, Wf1, bf1, wf2p, bf2p)
    return out[:, :2]


# speedup vs baseline: 259.7781x; 1.0107x over previous
import functools

import jax
import jax.numpy as jnp
from jax import lax
from jax.experimental import pallas as pl
from jax.experimental.pallas import tpu as pltpu
from jax.experimental.pallas import tpu_sc as plsc

NN = 10000
EE = 320000
GG = 128
NC = 2
NS = 16
NWK = NC * NS
NPT = 320
NP = NWK * NPT
EP = 327680
WG = EP // NWK
CG = 256
SCHUNK = 3200
STG = SCHUNK + 32

@functools.lru_cache(maxsize=None)
def _mesh():
    return plsc.VectorSubcoreMesh(core_axis_name="c", subcore_axis_name="s")


def _wid():
    return lax.axis_index("c") * NS + lax.axis_index("s")


def _sc_count_body(dst_hbm, counts_hbm, dbuf, cscr):
    wid = _wid()
    lo = wid * NPT
    hi = lo + NPT

    cscr[...] = jnp.zeros((16,), jnp.int32)
    lov = jnp.broadcast_to(lo, (16,))
    nptv = jnp.full((16,), NPT, jnp.uint32)
    onev = jnp.ones((16,), jnp.int32)
    zerov = jnp.zeros((16,), jnp.int32)

    def chunk(k, _):
        pltpu.sync_copy(dst_hbm.at[pl.ds(k * SCHUNK, SCHUNK)], dbuf)

        def inner(j, _2):
            d = dbuf[pl.ds(j * 16, 16)]
            m = plsc.bitcast(d - lov, jnp.uint32) < nptv
            cscr[...] = cscr[...] + jnp.where(m, onev, zerov)
            return 0

        lax.fori_loop(0, SCHUNK // 16, inner, 0)
        return 0

    lax.fori_loop(0, EE // SCHUNK, chunk, 0)
    pltpu.sync_copy(cscr, counts_hbm.at[wid])


@functools.lru_cache(maxsize=None)
def _sc_count():
    return pl.kernel(
        _sc_count_body, mesh=_mesh(),
        out_type=jax.ShapeDtypeStruct((NWK, 16), jnp.int32),
        scratch_types=[pltpu.VMEM((SCHUNK,), jnp.int32),
                       pltpu.VMEM((16,), jnp.int32)])


_GDN = lax.GatherDimensionNumbers(offset_dims=(), collapsed_slice_dims=(0,),
                                  start_index_map=(0,))


def _prefix16(x):
    lanev = lax.broadcasted_iota(jnp.int32, (16,), 0)
    zerov = jnp.zeros((16,), jnp.int32)
    onevv = jnp.ones((16,), jnp.int32)
    for sh in (1, 2, 4, 8):
        shv = jnp.full((16,), sh, jnp.int32)
        idxs = jnp.maximum(lanev - shv, zerov)
        stepm = jnp.minimum(idxs + (onevv - jnp.minimum(lanev, onevv)
                                    if sh == 1 else zerov), onevv)
        stepm = jnp.minimum(jnp.maximum(lanev - (shv - onevv), zerov), onevv)
        shifted = lax.gather(x, idxs[:, None], _GDN, (1,),
                             mode=lax.GatherScatterMode.PROMISE_IN_BOUNDS)
        x = x + shifted * stepm
    return x


SPER = 4096


def _sc_bucket_body(dst_hbm, src_hbm, offs_hbm, dstp_hbm, srcp_hbm,
                    dbuf, sbuf, vidx, vvd, vvs, tmp, orow, shd, shs, sem):
    wid = _wid()
    sid = lax.axis_index("s")
    sidbase = sid * SPER
    sbm = pl.multiple_of(sidbase, 16)
    lo = wid * NPT
    pltpu.sync_copy(offs_hbm.at[wid], orow)
    base = pl.multiple_of(orow[...][0], 16)
    lov = jnp.broadcast_to(lo, (16,))
    nptv = jnp.full((16,), NPT, jnp.uint32)
    onev = jnp.ones((16,), jnp.int32)
    zerov = jnp.zeros((16,), jnp.int32)
    negv = jnp.full((16,), -1, jnp.int32)
    lanev = lax.broadcasted_iota(jnp.int32, (16,), 0)
    sb16 = jnp.broadcast_to(sidbase, (16,))
    trashv = sb16 + jnp.full((16,), SPER - 16, jnp.int32) + lanev

    def chunk(k, carry):
        rel0, wp0 = carry
        pltpu.sync_copy(dst_hbm.at[pl.ds(k * SCHUNK, SCHUNK)], dbuf)
        pltpu.sync_copy(src_hbm.at[pl.ds(k * SCHUNK, SCHUNK)], sbuf)

        def inner(j2, rel):
            for u in range(2):
                j = j2 * 2 + u
                jo = pl.multiple_of(j * 16, 16)
                d = dbuf[pl.ds(jo, 16)]
                s = sbuf[pl.ds(jo, 16)]
                m = plsc.bitcast(d - lov, jnp.uint32) < nptv
                m_i32 = jnp.where(m, onev, zerov)
                incl = _prefix16(m_i32)
                tgt = sb16 + jnp.broadcast_to(rel, (16,)) + (incl - m_i32)
                idx = trashv + m_i32 * (tgt - trashv)
                row = j // 8
                cb = pl.multiple_of((j % 8) * 16, 16)
                vidx[row, pl.ds(cb, 16)] = idx
                vvd[row, pl.ds(cb, 16)] = d
                vvs[row, pl.ds(cb, 16)] = s
                rel = rel + incl[15]
            return rel

        rel = lax.fori_loop(0, SCHUNK // 32, inner, rel0)

        def scat(r, _2):
            c1 = pltpu.async_copy(vvd.at[r], shd.at[vidx.at[r]], sem)
            c2 = pltpu.async_copy(vvs.at[r], shs.at[vidx.at[r]], sem)
            c1.wait()
            c2.wait()
            return 0

        lax.fori_loop(0, SCHUNK // 128, scat, 0)

        nb = rel // 16
        wp = pl.multiple_of(wp0, 16)

        def fl(b, _2):
            pltpu.sync_copy(shd.at[pl.ds(sbm + b * 16, 16)], tmp)
            pltpu.sync_copy(tmp, dstp_hbm.at[pl.ds(wp + b * 16, 16)])
            pltpu.sync_copy(shs.at[pl.ds(sbm + b * 16, 16)], tmp)
            pltpu.sync_copy(tmp, srcp_hbm.at[pl.ds(wp + b * 16, 16)])
            return 0

        lax.fori_loop(0, nb, fl, 0)
        mo = pl.multiple_of(nb * 16, 16)
        pltpu.sync_copy(shd.at[pl.ds(sbm + mo, 16)], tmp)
        pltpu.sync_copy(tmp, shd.at[pl.ds(sbm, 16)])
        pltpu.sync_copy(shs.at[pl.ds(sbm + mo, 16)], tmp)
        pltpu.sync_copy(tmp, shs.at[pl.ds(sbm, 16)])
        return (rel - nb * 16, wp + nb * 16)

    rel, wp0 = lax.fori_loop(0, EE // SCHUNK, chunk, (jnp.int32(0), base))
    wp = pl.multiple_of(wp0, 16)
    relv = jnp.broadcast_to(rel, (16,))
    pltpu.sync_copy(shd.at[pl.ds(sbm, 16)], tmp)
    tmp[...] = jnp.where(lanev < relv, tmp[...], negv)
    pltpu.sync_copy(tmp, dstp_hbm.at[pl.ds(wp, 16)])
    pltpu.sync_copy(shs.at[pl.ds(sbm, 16)], tmp)
    relv2 = jnp.broadcast_to(rel, (16,))
    tmp[...] = jnp.where(lanev < relv2, tmp[...], negv)
    pltpu.sync_copy(tmp, srcp_hbm.at[pl.ds(wp, 16)])


@functools.lru_cache(maxsize=None)
def _sc_bucket():
    return pl.kernel(
        _sc_bucket_body, mesh=_mesh(),
        out_type=(jax.ShapeDtypeStruct((EP,), jnp.int32),
                  jax.ShapeDtypeStruct((EP,), jnp.int32)),
        scratch_types=[pltpu.VMEM((SCHUNK,), jnp.int32),
                       pltpu.VMEM((SCHUNK,), jnp.int32),
                       pltpu.VMEM((SCHUNK // 128, 128), jnp.int32),
                       pltpu.VMEM((SCHUNK // 128, 128), jnp.int32),
                       pltpu.VMEM((SCHUNK // 128, 128), jnp.int32),
                       pltpu.VMEM((16,), jnp.int32),
                       pltpu.VMEM((16,), jnp.int32),
                       pltpu.VMEM_SHARED((NS * SPER,), jnp.int32),
                       pltpu.VMEM_SHARED((NS * SPER,), jnp.int32),
                       pltpu.SemaphoreType.DMA])


@functools.lru_cache(maxsize=None)
def _make_sc_gather(nrows):
    def gather(pq_hbm, dstp2, srcp2, v_hbm, di, si, pb, qb, vb, sp, sq):
        wid = _wid()
        ziv = jnp.zeros((16,), jnp.int32)
        mxv = jnp.full((16,), nrows - 1, jnp.int32)
        zfv = jnp.zeros((16,), jnp.float32)

        def chunk(k, _):
            r0 = wid * WG + k * CG
            pltpu.sync_copy(dstp2.at[pl.ds(wid * (WG // 128) + k * 2, 2)], di)
            pltpu.sync_copy(srcp2.at[pl.ds(wid * (WG // 128) + k * 2, 2)], si)
            for j in range(2):
                for c in range(8):
                    dvv = di[j, pl.ds(c * 16, 16)]
                    di[j, pl.ds(c * 16, 16)] = jnp.minimum(
                        jnp.maximum(dvv, ziv), mxv)
                    svv = si[j, pl.ds(c * 16, 16)]
                    si[j, pl.ds(c * 16, 16)] = jnp.minimum(
                        jnp.maximum(svv, ziv), mxv)
            cps = [pltpu.async_copy(pq_hbm.at[di.at[j]],
                                    pb.at[pl.ds(j * 128, 128)], sp)
                   for j in range(2)]
            cqs = [pltpu.async_copy(pq_hbm.at[si.at[j]],
                                    qb.at[pl.ds(j * 128, 128)], sq)
                   for j in range(2)]
            for cp in cps:
                cp.wait()
            for cq in cqs:
                cq.wait()

            def row(r, _2):
                for c in range(4):
                    vb[r, pl.ds(c * 16, 16)] = jnp.maximum(
                        pb[r, pl.ds(c * 16, 16)]
                        + qb[r, pl.ds(64 + c * 16, 16)], zfv)
                return 0

            lax.fori_loop(0, CG, row, 0)
            pltpu.sync_copy(vb, v_hbm.at[pl.ds(r0, CG)])
            return 0

        lax.fori_loop(0, WG // CG, chunk, 0)

    return pl.kernel(
        gather, mesh=_mesh(),
        out_type=jax.ShapeDtypeStruct((EP, 64), jnp.float32),
        scratch_types=[pltpu.VMEM((2, 128), jnp.int32),
                       pltpu.VMEM((2, 128), jnp.int32),
                       pltpu.VMEM((CG, 128), jnp.float32),
                       pltpu.VMEM((CG, 128), jnp.float32),
                       pltpu.VMEM((CG, 64), jnp.float32),
                       pltpu.SemaphoreType.DMA,
                       pltpu.SemaphoreType.DMA])


def _sc_max_body(m_hbm, dstp_hbm, offs_hbm, cnts_hbm, h_hbm,
                 acc, mb, dv, offv, cntv):
    wid = _wid()
    lo = wid * NPT
    pltpu.sync_copy(offs_hbm.at[wid], offv)
    pltpu.sync_copy(cnts_hbm.at[wid], cntv)
    off = pl.multiple_of(offv[...][0], 16)
    cnt = cntv[...][0]

    def zr(r, _):
        for c in range(4):
            acc[r, pl.ds(c * 16, 16)] = jnp.zeros((16,), jnp.float32)
        return 0

    lax.fori_loop(0, NPT + 8, zr, 0)
    nch = (cnt + CG - 1) // CG

    def chunk(k, _):
        pltpu.sync_copy(dstp_hbm.at[pl.ds(off + k * CG, CG)], dv)
        pltpu.sync_copy(m_hbm.at[pl.ds(off + k * CG, CG)], mb)
        ne = jnp.minimum(CG, cnt - k * CG)
        nb16 = (ne + 15) // 16

        lov = jnp.broadcast_to(lo, (16,))

        def blk(b, _2):
            dvec = dv[pl.ds(b * 16, 16)] - lov
            for lane in range(16):
                dl = dvec[lane]
                dls = jnp.where(dl >= 0, dl, NPT)
                e = b * 16 + lane
                for c in range(4):
                    acc[dls, pl.ds(c * 16, 16)] = jnp.maximum(
                        acc[dls, pl.ds(c * 16, 16)], mb[e, pl.ds(c * 16, 16)])
            return 0

        lax.fori_loop(0, nb16, blk, 0)
        return 0

    lax.fori_loop(0, nch, chunk, 0)
    pltpu.sync_copy(acc.at[pl.ds(0, NPT)], h_hbm.at[pl.ds(lo, NPT)])


@functools.lru_cache(maxsize=None)
def _sc_max():
    return pl.kernel(
        _sc_max_body, mesh=_mesh(),
        out_type=jax.ShapeDtypeStruct((NP, 64), jnp.float32),
        scratch_types=[pltpu.VMEM((NPT + 8, 64), jnp.float32),
                       pltpu.VMEM((CG, 64), jnp.float32),
                       pltpu.VMEM((CG,), jnp.int32),
                       pltpu.VMEM((16,), jnp.int32),
                       pltpu.VMEM((16,), jnp.int32)])


def _proj_body(x_ref, wm_ref, wb_ref, b_ref, pq_ref):
    xb = x_ref[...]
    pq_ref[:, 0:64] = jnp.dot(xb, wm_ref[...],
                              preferred_element_type=jnp.float32) + b_ref[...]
    pq_ref[:, 64:128] = jnp.dot(xb, wb_ref[...],
                                preferred_element_type=jnp.float32)


def _tc_proj(xin, wm, wb, b1):
    rows, fdim = xin.shape
    br = 1000 if rows == NN else 1024
    return pl.pallas_call(
        _proj_body,
        grid=(rows // br,),
        in_specs=[pl.BlockSpec((br, fdim), lambda i: (i, 0)),
                  pl.BlockSpec((fdim, 64), lambda i: (0, 0)),
                  pl.BlockSpec((fdim, 64), lambda i: (0, 0)),
                  pl.BlockSpec((1, 64), lambda i: (0, 0))],
        out_specs=pl.BlockSpec((br, 128), lambda i: (i, 0)),
        out_shape=jax.ShapeDtypeStruct((rows, 128), jnp.float32),
    )(xin, wm, wb, b1.reshape(1, 64))


def _mm_body(v_ref, w_ref, b_ref, m_ref):
    m_ref[...] = jnp.dot(v_ref[...], w_ref[...],
                         preferred_element_type=jnp.float32) + b_ref[...]


def _tc_mm(v, w2, b2):
    br = 1024
    return pl.pallas_call(
        _mm_body,
        grid=(EP // br,),
        in_specs=[pl.BlockSpec((br, 64), lambda i: (i, 0)),
                  pl.BlockSpec((64, 64), lambda i: (0, 0)),
                  pl.BlockSpec((1, 64), lambda i: (0, 0))],
        out_specs=pl.BlockSpec((br, 64), lambda i: (i, 0)),
        out_shape=jax.ShapeDtypeStruct((EP, 64), jnp.float32),
    )(v, w2, b2.reshape(1, 64))


def _pool_body(h_ref, b_ref, wf1_ref, bf1_ref, wf2_ref, bf2_ref, o_ref,
               acc, cacc):
    i = pl.program_id(0)

    @pl.when(i == 0)
    def _():
        acc[...] = jnp.zeros_like(acc)
        cacc[...] = jnp.zeros_like(cacc)

    bvec = b_ref[0, 0, :].reshape(1, 1024)
    gids = lax.broadcasted_iota(jnp.int32, (GG, 1024), 0)
    mask = (gids == jnp.broadcast_to(bvec, (GG, 1024))).astype(jnp.float32)
    acc[...] += jnp.dot(mask, h_ref[...], preferred_element_type=jnp.float32)
    cacc[...] += jnp.sum(mask, axis=1, keepdims=True)

    @pl.when(i == (NP // 1024) - 1)
    def _():
        pooled = acc[...] / jnp.maximum(cacc[...], 1.0)
        o1 = jnp.maximum(
            jnp.dot(pooled, wf1_ref[...],
                    preferred_element_type=jnp.float32) + bf1_ref[...], 0.0)
        o_ref[...] = jnp.dot(o1, wf2_ref[...],
                             preferred_element_type=jnp.float32) + bf2_ref[...]


def _tc_pool(h3, batch3, wf1, bf1, wf2p, bf2p):
    return pl.pallas_call(
        _pool_body,
        grid=(NP // 1024,),
        in_specs=[pl.BlockSpec((1024, 64), lambda i: (i, 0)),
                  pl.BlockSpec((1, 1, 1024), lambda i: (i, 0, 0)),
                  pl.BlockSpec((64, 64), lambda i: (0, 0)),
                  pl.BlockSpec((1, 64), lambda i: (0, 0)),
                  pl.BlockSpec((64, 128), lambda i: (0, 0)),
                  pl.BlockSpec((1, 128), lambda i: (0, 0))],
        out_specs=pl.BlockSpec((GG, 128), lambda i: (0, 0)),
        out_shape=jax.ShapeDtypeStruct((GG, 128), jnp.float32),
        scratch_shapes=[pltpu.VMEM((GG, 64), jnp.float32),
                        pltpu.VMEM((GG, 1), jnp.float32)],
    )(h3, batch3, wf1, bf1.reshape(1, 64), wf2p, bf2p.reshape(1, 128))


def kernel(x, edge_index, batch, W1a, b1a, W2a, b2a, W1b, b1b, W2b, b2b,
           W1c, b1c, W2c, b2c, Wf1, bf1, Wf2, bf2):
    src = edge_index[0]
    dst = edge_index[1]

    counts2d = _sc_count()(dst)
    cnt32 = jnp.sum(counts2d, axis=1)
    pc = ((cnt32 + 15) // 16 + 1) * 16
    offs = jnp.concatenate(
        [jnp.zeros((1,), jnp.int32),
         jnp.cumsum(pc)[:-1].astype(jnp.int32)])
    offs_b = jnp.broadcast_to(offs[:, None], (NWK, 16))
    cnts_b = jnp.broadcast_to(cnt32[:, None], (NWK, 16)).astype(jnp.int32)
    dst_p, src_p = _sc_bucket()(dst, src, offs_b)
    dst_p2 = dst_p.reshape(EP // 128, 128)
    src_p2 = src_p.reshape(EP // 128, 128)

    def layer(xin, w1, b1, w2, b2, gather_fn):
        fdim = xin.shape[1]
        wm = w1[:fdim] - w1[fdim:]
        wb = w1[fdim:]
        pq = _tc_proj(xin, wm, wb, b1)
        v = gather_fn(pq, dst_p2, src_p2)
        m = _tc_mm(v, w2, b2)
        return _sc_max()(m, dst_p, offs_b, cnts_b)

    h1 = layer(x, W1a, b1a, W2a, b2a, _make_sc_gather(NN))
    h2 = layer(h1, W1b, b1b, W2b, b2b, _make_sc_gather(NP))
    h3 = layer(h2, W1c, b1c, W2c, b2c, _make_sc_gather(NP))

    batch_p = jnp.concatenate(
        [batch, jnp.full((NP - NN,), GG, jnp.int32)]).reshape(NP // 1024, 1, 1024)
    wf2p = jnp.zeros((64, 128), jnp.float32).at[:, :2].set(Wf2)
    bf2p = jnp.zeros((128,), jnp.float32).at[:2].set(bf2)
    out = _tc_pool(h3, batch_p, Wf1, bf1, wf2p, bf2p)
    return out[:, :2]
